# Initial kernel scaffold; baseline (speedup 1.0000x reference)
import jax
import jax.numpy as jnp
from jax.experimental import pallas as pl


def kernel(x, U_w, U_b, V_w, V_b, bn_w, bn_b):
    raise NotImplementedError("write your pallas kernel here")


# speedup vs baseline: 9.2942x; 9.2942x over previous
import jax
import jax.numpy as jnp
from jax.experimental import pallas as pl

_BLK1 = 8
_BLK2 = 32


def _dot_t(a, b):
    return jax.lax.dot_general(a, b, (((1,), (1,)), ((), ())),
                               preferred_element_type=jnp.float32)


def _pass1_body(x_ref, uw_ref, ub_ref, vw_ref, vb_ref, h_ref, s_ref, s2_ref):
    B, n, c = x_ref.shape
    x = x_ref[...]
    xf = x.reshape(B * n, c)
    vx = (_dot_t(xf, vw_ref[...]) + vb_ref[...]).reshape(B, n, c)
    ux = (_dot_t(xf, uw_ref[...]) + ub_ref[...]).reshape(B, n, c)

    s_part = jnp.zeros((n, 1), jnp.float32)
    s2_part = jnp.zeros((n, 1), jnp.float32)
    for i in range(B):
        xs = x[i]
        si = _dot_t(xs, xs)

        cur = jnp.max(si, axis=1, keepdims=True)
        cnt = jnp.sum((si == cur).astype(jnp.float32), axis=1, keepdims=True)
        thr = cur
        for _ in range(3):
            nxt = jnp.max(jnp.where(si < cur, si, -jnp.inf), axis=1,
                          keepdims=True)
            cnt_n = jnp.sum((si == nxt).astype(jnp.float32), axis=1,
                            keepdims=True)
            need = cnt < 4.0
            thr = jnp.where(need, nxt, thr)
            cnt = jnp.where(need, cnt + cnt_n, cnt)
            cur = nxt

        adj = (si >= thr).astype(jnp.float32)
        dinv = jax.lax.rsqrt(jnp.sum(adj, axis=1, keepdims=True))
        agg = dinv * jnp.dot(adj, dinv * vx[i],
                             preferred_element_type=jnp.float32)
        h = agg + ux[i]
        h_ref[i] = h
        s_part = s_part + jnp.sum(h, axis=1, keepdims=True)
        s2_part = s2_part + jnp.sum(h * h, axis=1, keepdims=True)

    @pl.when(pl.program_id(0) == 0)
    def _init():
        s_ref[...] = jnp.zeros_like(s_ref)
        s2_ref[...] = jnp.zeros_like(s2_ref)

    s_ref[...] += s_part
    s2_ref[...] += s2_part


def _pass2_body(x_ref, h_ref, sc_ref, sh_ref, o_ref):
    _, n, _ = x_ref.shape
    sc = sc_ref[...].reshape(1, n, 1)
    sh = sh_ref[...].reshape(1, n, 1)
    o_ref[...] = jnp.maximum(x_ref[...] + h_ref[...] * sc + sh, 0.0)


def kernel(x, U_w, U_b, V_w, V_b, bn_w, bn_b):
    b, n, c = x.shape
    ub = U_b.reshape(1, c)
    vb = V_b.reshape(1, c)

    h, s, s2 = pl.pallas_call(
        _pass1_body,
        grid=(b // _BLK1,),
        in_specs=[
            pl.BlockSpec((_BLK1, n, c), lambda i: (i, 0, 0)),
            pl.BlockSpec((c, c), lambda i: (0, 0)),
            pl.BlockSpec((1, c), lambda i: (0, 0)),
            pl.BlockSpec((c, c), lambda i: (0, 0)),
            pl.BlockSpec((1, c), lambda i: (0, 0)),
        ],
        out_specs=[
            pl.BlockSpec((_BLK1, n, c), lambda i: (i, 0, 0)),
            pl.BlockSpec((n, 1), lambda i: (0, 0)),
            pl.BlockSpec((n, 1), lambda i: (0, 0)),
        ],
        out_shape=[
            jax.ShapeDtypeStruct((b, n, c), jnp.float32),
            jax.ShapeDtypeStruct((n, 1), jnp.float32),
            jax.ShapeDtypeStruct((n, 1), jnp.float32),
        ],
    )(x, U_w, ub, V_w, vb)

    denom = float(b * c)
    mean = s / denom
    var = s2 / denom - mean * mean
    scale = bn_w.reshape(n, 1) * jax.lax.rsqrt(var + 1e-5)
    shift = bn_b.reshape(n, 1) - mean * scale

    out = pl.pallas_call(
        _pass2_body,
        grid=(b // _BLK2,),
        in_specs=[
            pl.BlockSpec((_BLK2, n, c), lambda i: (i, 0, 0)),
            pl.BlockSpec((_BLK2, n, c), lambda i: (i, 0, 0)),
            pl.BlockSpec((n, 1), lambda i: (0, 0)),
            pl.BlockSpec((n, 1), lambda i: (0, 0)),
        ],
        out_specs=pl.BlockSpec((_BLK2, n, c), lambda i: (i, 0, 0)),
        out_shape=jax.ShapeDtypeStruct((b, n, c), jnp.float32),
    )(x, h, scale, shift)
    return out


# speedup vs baseline: 9.3492x; 1.0059x over previous
import jax
import jax.numpy as jnp
from jax.experimental import pallas as pl

_BLK1 = 8
_BLK2 = 32


def _dot_t(a, b):
    return jax.lax.dot_general(a, b, (((1,), (1,)), ((), ())),
                               preferred_element_type=jnp.float32)


def _pass1_body(x_ref, uw_ref, ub_ref, vw_ref, vb_ref, h_ref, s_ref, s2_ref):
    B, n, c = x_ref.shape
    x = x_ref[...]
    xf = x.reshape(B * n, c)
    vx = (_dot_t(xf, vw_ref[...]) + vb_ref[...]).reshape(B, n, c)
    ux = (_dot_t(xf, uw_ref[...]) + ub_ref[...]).reshape(B, n, c)

    s_part = jnp.zeros((n, 1), jnp.float32)
    s2_part = jnp.zeros((n, 1), jnp.float32)
    for i in range(B):
        xs = x[i]
        si = _dot_t(xs, xs)

        neg = jnp.float32(-jnp.inf)
        m1 = jnp.max(si, axis=1, keepdims=True)
        t = jnp.where(si < m1, si, neg)
        m2 = jnp.max(t, axis=1, keepdims=True)
        t = jnp.where(t < m2, t, neg)
        m3 = jnp.max(t, axis=1, keepdims=True)
        t = jnp.where(t < m3, t, neg)
        m4 = jnp.max(t, axis=1, keepdims=True)
        c1 = jnp.sum((si >= m1).astype(jnp.float32), axis=1, keepdims=True)
        c2 = jnp.sum((si >= m2).astype(jnp.float32), axis=1, keepdims=True)
        c3 = jnp.sum((si >= m3).astype(jnp.float32), axis=1, keepdims=True)
        thr = jnp.where(c1 >= 4.0, m1,
                        jnp.where(c2 >= 4.0, m2,
                                  jnp.where(c3 >= 4.0, m3, m4)))

        adj = (si >= thr).astype(jnp.float32)
        dinv = jax.lax.rsqrt(jnp.sum(adj, axis=1, keepdims=True))
        agg = dinv * jnp.dot(adj, dinv * vx[i],
                             preferred_element_type=jnp.float32)
        h = agg + ux[i]
        h_ref[i] = h
        s_part = s_part + jnp.sum(h, axis=1, keepdims=True)
        s2_part = s2_part + jnp.sum(h * h, axis=1, keepdims=True)

    @pl.when(pl.program_id(0) == 0)
    def _init():
        s_ref[...] = jnp.zeros_like(s_ref)
        s2_ref[...] = jnp.zeros_like(s2_ref)

    s_ref[...] += s_part
    s2_ref[...] += s2_part


def _pass2_body(x_ref, h_ref, sc_ref, sh_ref, o_ref):
    _, n, _ = x_ref.shape
    sc = sc_ref[...].reshape(1, n, 1)
    sh = sh_ref[...].reshape(1, n, 1)
    o_ref[...] = jnp.maximum(x_ref[...] + h_ref[...] * sc + sh, 0.0)


def kernel(x, U_w, U_b, V_w, V_b, bn_w, bn_b):
    b, n, c = x.shape
    ub = U_b.reshape(1, c)
    vb = V_b.reshape(1, c)

    h, s, s2 = pl.pallas_call(
        _pass1_body,
        grid=(b // _BLK1,),
        in_specs=[
            pl.BlockSpec((_BLK1, n, c), lambda i: (i, 0, 0)),
            pl.BlockSpec((c, c), lambda i: (0, 0)),
            pl.BlockSpec((1, c), lambda i: (0, 0)),
            pl.BlockSpec((c, c), lambda i: (0, 0)),
            pl.BlockSpec((1, c), lambda i: (0, 0)),
        ],
        out_specs=[
            pl.BlockSpec((_BLK1, n, c), lambda i: (i, 0, 0)),
            pl.BlockSpec((n, 1), lambda i: (0, 0)),
            pl.BlockSpec((n, 1), lambda i: (0, 0)),
        ],
        out_shape=[
            jax.ShapeDtypeStruct((b, n, c), jnp.float32),
            jax.ShapeDtypeStruct((n, 1), jnp.float32),
            jax.ShapeDtypeStruct((n, 1), jnp.float32),
        ],
    )(x, U_w, ub, V_w, vb)

    denom = float(b * c)
    mean = s / denom
    var = s2 / denom - mean * mean
    scale = bn_w.reshape(n, 1) * jax.lax.rsqrt(var + 1e-5)
    shift = bn_b.reshape(n, 1) - mean * scale

    out = pl.pallas_call(
        _pass2_body,
        grid=(b // _BLK2,),
        in_specs=[
            pl.BlockSpec((_BLK2, n, c), lambda i: (i, 0, 0)),
            pl.BlockSpec((_BLK2, n, c), lambda i: (i, 0, 0)),
            pl.BlockSpec((n, 1), lambda i: (0, 0)),
            pl.BlockSpec((n, 1), lambda i: (0, 0)),
        ],
        out_specs=pl.BlockSpec((_BLK2, n, c), lambda i: (i, 0, 0)),
        out_shape=jax.ShapeDtypeStruct((b, n, c), jnp.float32),
    )(x, h, scale, shift)
    return out
